# baseline (speedup 1.0000x reference)
import functools
import math

import jax
import jax.numpy as jnp
from jax import lax
from jax.experimental import pallas as pl
from jax.experimental.pallas import tpu as pltpu
from jax.experimental.pallas import tpu_sc as plsc

D = 64
SCALE = math.sqrt(D)
N_PAIRS = 500000

NC = 2
NS = 16
NW = NC * NS
L = 16

CHUNK = 128
NBUF = 2


def _emb_kernel(n_chunks, cpp, idx_hbm, lut_hbm, out_hbm,
                idx_v, rows_v, tbuf_v, gsems, wsems):
    wid = lax.axis_index("s") * NC + lax.axis_index("c")
    pltpu.sync_copy(idx_hbm.at[wid], idx_v)

    iota = lax.iota(jnp.int32, L)
    rvecs = [iota + g * L for g in range(CHUNK // L)]

    def start_gather(j, b):
        pltpu.async_copy(lut_hbm.at[idx_v.at[j]], rows_v.at[b], gsems.at[b])

    def out_dst(jj):
        g = wid * n_chunks + jj
        p = g // cpp
        bt = lax.rem(g, cpp)
        return out_hbm.at[p, :, bt]

    for b in range(NBUF):
        start_gather(b, b)

    def body(j):
        for b in range(NBUF):
            jj = j + b
            pltpu.make_async_copy(lut_hbm.at[idx_v.at[jj]],
                                  rows_v.at[b], gsems.at[b]).wait()
            @pl.when(jj >= NBUF)
            def _():
                pltpu.make_async_copy(tbuf_v.at[b], out_dst(jj - NBUF),
                                      wsems.at[b]).wait()

            for dd in range(D):
                cvec = jnp.full((L,), dd, jnp.int32)
                for g in range(CHUNK // L):
                    v = plsc.load_gather(rows_v.at[b], [rvecs[g], cvec])
                    tbuf_v[b, dd // 8, dd % 8, pl.ds(g * L, L)] = v * SCALE

            pltpu.async_copy(tbuf_v.at[b], out_dst(jj), wsems.at[b])
            @pl.when(jj + NBUF < n_chunks)
            def _():
                start_gather(jj + NBUF, b)

    pl.loop(0, n_chunks, step=NBUF)(body)

    for b in range(NBUF):
        jj = n_chunks - NBUF + b
        pltpu.make_async_copy(tbuf_v.at[b], out_dst(jj), wsems.at[b]).wait()


@jax.jit
def kernel(x, lut):
    n_batch, n_pos = x.shape
    B = n_batch * n_pos
    n_chunks = B // (NW * CHUNK)
    cpp = n_batch // CHUNK
    idx = x.T.astype(jnp.int32).reshape(NW, n_chunks, CHUNK)
    lut2 = jax.lax.optimization_barrier(lut.reshape(N_PAIRS, 2 * D))
    lut_rm = lut2.reshape(lut.shape)

    mesh = plsc.VectorSubcoreMesh(core_axis_name="c", subcore_axis_name="s")
    run = pl.kernel(
        functools.partial(_emb_kernel, n_chunks, cpp),
        out_type=jax.ShapeDtypeStruct((n_pos, 8, cpp, 8, CHUNK), jnp.float32),
        mesh=mesh,
        scratch_types=[
            pltpu.VMEM((n_chunks, CHUNK), jnp.int32),
            pltpu.VMEM((NBUF, CHUNK, D), jnp.float32),
            pltpu.VMEM((NBUF, 8, 8, CHUNK), jnp.float32),
            pltpu.SemaphoreType.DMA((NBUF,)),
            pltpu.SemaphoreType.DMA((NBUF,)),
        ],
        compiler_params=pltpu.CompilerParams(use_tc_tiling_on_sc=False,
                                             needs_layout_passes=False),
    )
    t5 = run(idx, lut_rm)
    o = t5.transpose(0, 1, 3, 2, 4).reshape(n_pos, D, n_batch)
    return o.transpose(2, 0, 1)


# speedup vs baseline: 1.3831x; 1.3831x over previous
import functools
import math

import jax
import jax.numpy as jnp
from jax import lax
from jax.experimental import pallas as pl
from jax.experimental.pallas import tpu as pltpu
from jax.experimental.pallas import tpu_sc as plsc

D = 64
SCALE = math.sqrt(D)
N_PAIRS = 500000

NC = 2
NS = 16
NW = NC * NS
L = 16

CHUNK = 128
NBUF = 2
RUNROLL = 4


def _emb_kernel(n_chunks, cpp, idx_hbm, lut_hbm, out_hbm,
                idx_v, rows_v, cbuf_v, gsems, wsems):
    wid = lax.axis_index("s") * NC + lax.axis_index("c")
    pltpu.sync_copy(idx_hbm.at[wid], idx_v)

    def start_gather(j, b):
        pltpu.async_copy(lut_hbm.at[idx_v.at[j]], rows_v.at[b], gsems.at[b])

    for b in range(NBUF):
        start_gather(b, b)

    def body(j):
        for b in range(NBUF):
            jj = j + b
            pltpu.make_async_copy(lut_hbm.at[idx_v.at[jj]],
                                  rows_v.at[b], gsems.at[b]).wait()
            def scale(i):
                t = i // 8
                for r in range(8):
                    for d in range(D // L):
                        v = rows_v[b, i + r, pl.ds(d * L, L)]
                        cbuf_v[b, t, r, pl.ds(d * L, L)] = v * SCALE
            pl.loop(0, CHUNK, step=8)(scale)
            g = wid * n_chunks + jj
            dst = out_hbm.at[g // cpp, pl.ds(lax.rem(g, cpp) * (CHUNK // 8),
                                             CHUNK // 8)]
            pltpu.async_copy(cbuf_v.at[b], dst, wsems.at[b])
            @pl.when(jj + NBUF < n_chunks)
            def _():
                pltpu.make_async_copy(cbuf_v.at[b], dst, wsems.at[b]).wait()
                start_gather(jj + NBUF, b)

    pl.loop(0, n_chunks, step=NBUF)(body)

    for b in range(NBUF):
        jj = n_chunks - NBUF + b
        g = wid * n_chunks + jj
        pltpu.make_async_copy(
            cbuf_v.at[b],
            out_hbm.at[g // cpp, pl.ds(lax.rem(g, cpp) * (CHUNK // 8),
                                       CHUNK // 8)],
            wsems.at[b]).wait()


@jax.jit
def kernel(x, lut):
    n_batch, n_pos = x.shape
    B = n_batch * n_pos
    n_chunks = B // (NW * CHUNK)
    cpp = n_batch // CHUNK
    idx = x.T.astype(jnp.int32).reshape(NW, n_chunks, CHUNK)
    lut2 = jax.lax.optimization_barrier(lut.reshape(N_PAIRS, 2 * D))
    lut_rm = lut2.reshape(lut.shape)

    mesh = plsc.VectorSubcoreMesh(core_axis_name="c", subcore_axis_name="s")
    run = pl.kernel(
        functools.partial(_emb_kernel, n_chunks, cpp),
        out_type=jax.ShapeDtypeStruct(
            (n_pos, n_batch // 8, 8, CHUNK), jnp.float32),
        mesh=mesh,
        scratch_types=[
            pltpu.VMEM((n_chunks, CHUNK), jnp.int32),
            pltpu.VMEM((NBUF, CHUNK, D), jnp.float32),
            pltpu.VMEM((NBUF, CHUNK // 8, 8, CHUNK), jnp.float32),
            pltpu.SemaphoreType.DMA((NBUF,)),
            pltpu.SemaphoreType.DMA((NBUF,)),
        ],
        compiler_params=pltpu.CompilerParams(use_tc_tiling_on_sc=False),
    )
    out = run(idx, lut_rm)
    o = out[:, :, :, :D].reshape(n_pos, n_batch, D)
    return o.transpose(1, 0, 2)


# speedup vs baseline: 1.9517x; 1.4111x over previous
import functools
import math

import jax
import jax.numpy as jnp
from jax import lax
from jax.experimental import pallas as pl
from jax.experimental.pallas import tpu as pltpu
from jax.experimental.pallas import tpu_sc as plsc

D = 64
SCALE = math.sqrt(D)
N_PAIRS = 500000

NC = 2
NS = 16
NW = NC * NS
L = 16

CHUNK = 128
NBUF = 2
RUNROLL = 4


def _emb_kernel(n_chunks, cpp, idx_hbm, lut_hbm, out_hbm,
                idx_v, rows_v, tbuf_v, gsems, wsems):
    wid = lax.axis_index("s") * NC + lax.axis_index("c")
    pltpu.sync_copy(idx_hbm.at[wid], idx_v)

    iota = lax.iota(jnp.int32, L)
    dtv = [(iota + g * L) >> 3 for g in range(D // L)]
    drv = [(iota + g * L) & 7 for g in range(D // L)]

    def start_gather(j, b):
        pltpu.async_copy(lut_hbm.at[idx_v.at[j]], rows_v.at[b], gsems.at[b])

    def out_dst(jj):
        g = wid * n_chunks + jj
        return out_hbm.at[g // cpp, :, lax.rem(g, cpp)]

    for b in range(NBUF):
        start_gather(b, b)

    def body(j):
        for b in range(NBUF):
            jj = j + b
            pltpu.make_async_copy(lut_hbm.at[idx_v.at[jj]],
                                  rows_v.at[b], gsems.at[b]).wait()
            @pl.when(jj >= NBUF)
            def _():
                pltpu.make_async_copy(tbuf_v.at[b, :, :, pl.ds(0, CHUNK)],
                                      out_dst(jj - NBUF),
                                      wsems.at[b]).wait()
            def tstep(i):
                for r in range(RUNROLL):
                    br = i + r
                    colv = jnp.full((L,), br, jnp.int32)
                    for g in range(D // L):
                        v = rows_v[b, br, pl.ds(g * L, L)] * SCALE
                        plsc.store_scatter(tbuf_v.at[b],
                                           [dtv[g], drv[g], colv], v)
            pl.loop(0, CHUNK, step=RUNROLL)(tstep)
            pltpu.async_copy(tbuf_v.at[b, :, :, pl.ds(0, CHUNK)],
                             out_dst(jj), wsems.at[b])
            @pl.when(jj + NBUF < n_chunks)
            def _():
                start_gather(jj + NBUF, b)

    pl.loop(0, n_chunks, step=NBUF)(body)

    for b in range(NBUF):
        jj = n_chunks - NBUF + b
        pltpu.make_async_copy(tbuf_v.at[b, :, :, pl.ds(0, CHUNK)],
                              out_dst(jj), wsems.at[b]).wait()


@jax.jit
def kernel(x, lut):
    n_batch, n_pos = x.shape
    B = n_batch * n_pos
    n_chunks = B // (NW * CHUNK)
    cpp = n_batch // CHUNK
    idx = x.T.astype(jnp.int32).reshape(NW, n_chunks, CHUNK)
    lut2 = jax.lax.optimization_barrier(lut.reshape(N_PAIRS, 2 * D))
    lut_rm = lut2.reshape(lut.shape)

    mesh = plsc.VectorSubcoreMesh(core_axis_name="c", subcore_axis_name="s")
    run = pl.kernel(
        functools.partial(_emb_kernel, n_chunks, cpp),
        out_type=jax.ShapeDtypeStruct((n_pos, 8, cpp, 8, CHUNK),
                                      jnp.float32),
        mesh=mesh,
        scratch_types=[
            pltpu.VMEM((n_chunks, CHUNK), jnp.int32),
            pltpu.VMEM((NBUF, CHUNK, D), jnp.float32),
            pltpu.VMEM((NBUF, 8, 8, CHUNK + 1), jnp.float32),
            pltpu.SemaphoreType.DMA((NBUF,)),
            pltpu.SemaphoreType.DMA((NBUF,)),
        ],
        compiler_params=pltpu.CompilerParams(use_tc_tiling_on_sc=False,
                                             needs_layout_passes=False),
    )
    t5 = run(idx, lut_rm)
    o = t5.transpose(0, 1, 3, 2, 4).reshape(n_pos, D, n_batch)
    return o.transpose(2, 0, 1)
